# baseline (speedup 1.0000x reference)
import functools

import jax
import jax.numpy as jnp
from jax import lax
from jax.experimental import pallas as pl
from jax.experimental.pallas import tpu as pltpu
from jax.experimental.pallas import tpu_sc as plsc

N = 10000
E = 320000
D = 128
N_PAD = 10240
NC = 2
NS = 16
NW = NC * NS
RPT = N_PAD // NS
BM = 1000

EC = 80
EN = 128
E_PAD = NW * EN * EC
ZC = 80

DC = 80
DN = E // NW // DC
NBUF = 2
DG = DN // NBUF

_mesh = plsc.VectorSubcoreMesh(core_axis_name="c", subcore_axis_name="s")


@functools.partial(
    pl.kernel,
    mesh=_mesh,
    out_type=jax.ShapeDtypeStruct((NC, 2, N_PAD), jnp.float32),
    scratch_types=[
        pltpu.VMEM((DN, DC), jnp.int32),
        pltpu.VMEM((DN, DC), jnp.int32),
        pltpu.VMEM((DC,), jnp.float32),
        pltpu.VMEM_SHARED((N_PAD,), jnp.float32),
        pltpu.VMEM_SHARED((N_PAD,), jnp.float32),
        pltpu.SemaphoreType.DMA((NBUF,)),
        pltpu.SemaphoreType.DMA((NBUF,)),
    ],
)
def _deg_kernel(src_hbm, dst_hbm, ones_hbm, zeros_hbm, out_hbm,
                src_v, dst_v, ones_v, dego_sh, degi_sh, osem, isem):
    cid = lax.axis_index("c")
    sid = lax.axis_index("s")
    wid = sid * NC + cid
    pltpu.sync_copy(src_hbm.at[wid], src_v)
    pltpu.sync_copy(dst_hbm.at[wid], dst_v)
    pltpu.sync_copy(ones_hbm, ones_v)
    pltpu.sync_copy(zeros_hbm, dego_sh.at[pl.ds(sid * RPT, RPT)])
    pltpu.sync_copy(zeros_hbm, degi_sh.at[pl.ds(sid * RPT, RPT)])
    plsc.subcore_barrier()

    def start(j, b):
        pltpu.async_copy(ones_v, dego_sh.at[src_v.at[j]], osem.at[b], add=True)
        pltpu.async_copy(ones_v, degi_sh.at[dst_v.at[j]], isem.at[b], add=True)

    def drain(j, b):
        pltpu.make_async_copy(ones_v, dego_sh.at[src_v.at[j]], osem.at[b]).wait()
        pltpu.make_async_copy(ones_v, degi_sh.at[dst_v.at[j]], isem.at[b]).wait()

    for b in range(NBUF):
        start(b, b)

    def group(g, carry):
        for b in range(NBUF):
            j = g * NBUF + b
            drain(j, b)
            start(j + NBUF, b)
        return carry

    lax.fori_loop(0, DG - 1, group, 0)
    for b in range(NBUF):
        drain((DG - 1) * NBUF + b, b)
    start(DN - 1, 0)
    drain(DN - 1, 0)
    plsc.subcore_barrier()
    pltpu.sync_copy(dego_sh.at[pl.ds(sid * RPT, RPT)],
                    out_hbm.at[cid, 0, pl.ds(sid * RPT, RPT)])
    pltpu.sync_copy(degi_sh.at[pl.ds(sid * RPT, RPT)],
                    out_hbm.at[cid, 1, pl.ds(sid * RPT, RPT)])


@functools.partial(
    pl.kernel,
    mesh=_mesh,
    out_type=jax.ShapeDtypeStruct((NC, N_PAD, D), jnp.float32),
    scratch_types=[
        pltpu.VMEM((EN, EC), jnp.int32),
        pltpu.VMEM((EN, EC), jnp.int32),
        pltpu.VMEM((EC, D), jnp.float32),
        pltpu.VMEM_SHARED((N_PAD, D), jnp.float32),
        pltpu.SemaphoreType.DMA,
    ],
)
def _edge_kernel(h_hbm, src_hbm, dst_hbm, zrows_hbm, out_hbm,
                 src_v, dst_v, rows_v, acc_sh, gsem):
    cid = lax.axis_index("c")
    sid = lax.axis_index("s")
    wid = sid * NC + cid
    rbase = sid * RPT
    pltpu.sync_copy(src_hbm.at[wid], src_v)
    pltpu.sync_copy(dst_hbm.at[wid], dst_v)
    pltpu.sync_copy(zrows_hbm, rows_v.at[pl.ds(0, ZC)])
    for t in range(RPT // ZC):
        pltpu.sync_copy(rows_v.at[pl.ds(0, ZC)],
                        acc_sh.at[pl.ds(rbase + t * ZC, ZC)])
    plsc.subcore_barrier()

    def chunk(j, carry):
        pltpu.async_copy(h_hbm.at[src_v.at[j]], rows_v, gsem).wait()
        pltpu.sync_copy(rows_v, acc_sh.at[dst_v.at[j]], add=True)
        return carry

    lax.fori_loop(0, EN, chunk, 0)
    plsc.subcore_barrier()
    for t in range(RPT // ZC):
        pltpu.sync_copy(acc_sh.at[pl.ds(rbase + t * ZC, ZC)],
                        rows_v.at[pl.ds(0, ZC)])
        pltpu.sync_copy(rows_v.at[pl.ds(0, ZC)],
                        out_hbm.at[cid, pl.ds(rbase + t * ZC, ZC)])


def _norm_body(p_ref, out_ref):
    deg = p_ref[0] + p_ref[1] + 1.0
    out_ref[...] = lax.rsqrt(deg)


def _mm_scale_body(x_ref, w_ref, s_ref, o_ref):
    h = jnp.dot(x_ref[...], w_ref[...], preferred_element_type=jnp.float32)
    o_ref[...] = h * s_ref[...]


def _combine_mm_body(p_ref, hp_ref, ni_ref, b_ref, w_ref, no_ref, o_ref):
    agg = p_ref[0] + p_ref[1] + hp_ref[...]
    h = jnp.maximum(agg * ni_ref[...] + b_ref[...], 0.0)
    o_ref[...] = jnp.dot(h, w_ref[...], preferred_element_type=jnp.float32) * no_ref[...]


def _combine_final_body(p_ref, hp_ref, ni_ref, b_ref, o_ref):
    agg = p_ref[0] + p_ref[1] + hp_ref[...]
    o_ref[...] = agg * ni_ref[...] + b_ref[...]


def _norms(deg_p):
    return pl.pallas_call(
        _norm_body,
        out_shape=jax.ShapeDtypeStruct((2, N_PAD), jnp.float32),
    )(deg_p)


def _mm_scale(xv, W, s_col):
    return pl.pallas_call(
        _mm_scale_body,
        grid=(N // BM,),
        in_specs=[
            pl.BlockSpec((BM, D), lambda i: (i, 0)),
            pl.BlockSpec((D, D), lambda i: (0, 0)),
            pl.BlockSpec((BM, 1), lambda i: (i, 0)),
        ],
        out_specs=pl.BlockSpec((BM, D), lambda i: (i, 0)),
        out_shape=jax.ShapeDtypeStruct((N, D), jnp.float32),
    )(xv, W, s_col)


def _combine_mm(part, hp, ni_col, b_row, W, no_col):
    return pl.pallas_call(
        _combine_mm_body,
        grid=(N // BM,),
        in_specs=[
            pl.BlockSpec((NC, BM, D), lambda i: (0, i, 0)),
            pl.BlockSpec((BM, D), lambda i: (i, 0)),
            pl.BlockSpec((BM, 1), lambda i: (i, 0)),
            pl.BlockSpec((1, D), lambda i: (0, 0)),
            pl.BlockSpec((D, D), lambda i: (0, 0)),
            pl.BlockSpec((BM, 1), lambda i: (i, 0)),
        ],
        out_specs=pl.BlockSpec((BM, D), lambda i: (i, 0)),
        out_shape=jax.ShapeDtypeStruct((N, D), jnp.float32),
    )(part, hp, ni_col, b_row, W, no_col)


def _combine_final(part, hp, ni_col, b_row):
    return pl.pallas_call(
        _combine_final_body,
        grid=(N // BM,),
        in_specs=[
            pl.BlockSpec((NC, BM, D), lambda i: (0, i, 0)),
            pl.BlockSpec((BM, D), lambda i: (i, 0)),
            pl.BlockSpec((BM, 1), lambda i: (i, 0)),
            pl.BlockSpec((1, D), lambda i: (0, 0)),
        ],
        out_specs=pl.BlockSpec((BM, D), lambda i: (i, 0)),
        out_shape=jax.ShapeDtypeStruct((N, D), jnp.float32),
    )(part, hp, ni_col, b_row)


def kernel(x, edge_index, W0, b0, W1, b1):
    src = edge_index[0]
    dst = edge_index[1]
    src_d = src.reshape(NW, DN, DC)
    dst_d = dst.reshape(NW, DN, DC)
    ppt = (E_PAD - E) // NW
    src_pad = jnp.zeros((NW, ppt), jnp.int32)
    dst_pad = jnp.broadcast_to(N + jnp.arange(ppt, dtype=jnp.int32), (NW, ppt))
    src_e = jnp.concatenate([src.reshape(NW, E // NW), src_pad], axis=1).reshape(NW, EN, EC)
    dst_e = jnp.concatenate([dst.reshape(NW, E // NW), dst_pad], axis=1).reshape(NW, EN, EC)
    ones_c = jnp.ones((DC,), jnp.float32)
    zeros_r = jnp.zeros((RPT,), jnp.float32)
    zrows = jnp.zeros((ZC, D), jnp.float32)

    deg_p = _deg_kernel(src_d, dst_d, ones_c, zeros_r)
    norms = _norms(deg_p)
    no_col = norms[0, :N].reshape(N, 1)
    ni_col = norms[1, :N].reshape(N, 1)

    h0p = _mm_scale(x, W0, no_col)
    part0 = _edge_kernel(h0p, src_e, dst_e, zrows)
    h1p = _combine_mm(part0, h0p, ni_col, b0.reshape(1, D), W1, no_col)
    part1 = _edge_kernel(h1p, src_e, dst_e, zrows)
    return _combine_final(part1, h1p, ni_col, b1.reshape(1, D))


# speedup vs baseline: 2.1089x; 2.1089x over previous
import functools

import jax
import jax.numpy as jnp
from jax import lax
from jax.experimental import pallas as pl
from jax.experimental.pallas import tpu as pltpu
from jax.experimental.pallas import tpu_sc as plsc

N = 10000
E = 320000
D = 128
N_PAD = 10240
NC = 2
NS = 16
NW = NC * NS
RPT = N_PAD // NS
BM = 1000

EC = 80
EN = 125
E_PAD = NW * EN * EC
ZC = 80

DC = 80
DN = E // NW // DC
NBUF = 2
DG = DN // NBUF

_mesh = plsc.VectorSubcoreMesh(core_axis_name="c", subcore_axis_name="s")


@functools.partial(
    pl.kernel,
    mesh=_mesh,
    out_type=jax.ShapeDtypeStruct((NC, 2, N_PAD), jnp.float32),
    scratch_types=[
        pltpu.VMEM((DN, DC), jnp.int32),
        pltpu.VMEM((DN, DC), jnp.int32),
        pltpu.VMEM((DC,), jnp.float32),
        pltpu.VMEM_SHARED((N_PAD,), jnp.float32),
        pltpu.VMEM_SHARED((N_PAD,), jnp.float32),
        pltpu.SemaphoreType.DMA((NBUF,)),
        pltpu.SemaphoreType.DMA((NBUF,)),
    ],
)
def _deg_kernel(src_hbm, dst_hbm, ones_hbm, zeros_hbm, out_hbm,
                src_v, dst_v, ones_v, dego_sh, degi_sh, osem, isem):
    cid = lax.axis_index("c")
    sid = lax.axis_index("s")
    wid = sid * NC + cid
    pltpu.sync_copy(src_hbm.at[wid], src_v)
    pltpu.sync_copy(dst_hbm.at[wid], dst_v)
    pltpu.sync_copy(ones_hbm, ones_v)
    pltpu.sync_copy(zeros_hbm, dego_sh.at[pl.ds(sid * RPT, RPT)])
    pltpu.sync_copy(zeros_hbm, degi_sh.at[pl.ds(sid * RPT, RPT)])
    plsc.subcore_barrier()

    def start(j, b):
        pltpu.async_copy(ones_v, dego_sh.at[src_v.at[j]], osem.at[b], add=True)
        pltpu.async_copy(ones_v, degi_sh.at[dst_v.at[j]], isem.at[b], add=True)

    def drain(j, b):
        pltpu.make_async_copy(ones_v, dego_sh.at[src_v.at[j]], osem.at[b]).wait()
        pltpu.make_async_copy(ones_v, degi_sh.at[dst_v.at[j]], isem.at[b]).wait()

    for b in range(NBUF):
        start(b, b)

    def group(g, carry):
        for b in range(NBUF):
            j = g * NBUF + b
            drain(j, b)
            start(j + NBUF, b)
        return carry

    lax.fori_loop(0, DG - 1, group, 0)
    for b in range(NBUF):
        drain((DG - 1) * NBUF + b, b)
    start(DN - 1, 0)
    drain(DN - 1, 0)
    plsc.subcore_barrier()
    pltpu.sync_copy(dego_sh.at[pl.ds(sid * RPT, RPT)],
                    out_hbm.at[cid, 0, pl.ds(sid * RPT, RPT)])
    pltpu.sync_copy(degi_sh.at[pl.ds(sid * RPT, RPT)],
                    out_hbm.at[cid, 1, pl.ds(sid * RPT, RPT)])


@functools.partial(
    pl.kernel,
    mesh=_mesh,
    out_type=jax.ShapeDtypeStruct((NC, N_PAD, D), jnp.float32),
    scratch_types=[
        pltpu.VMEM((EN, EC), jnp.int32),
        pltpu.VMEM((EN, EC), jnp.int32),
        pltpu.VMEM((EC, D), jnp.float32),
        pltpu.VMEM_SHARED((N_PAD, D), jnp.float32),
        pltpu.SemaphoreType.DMA,
    ],
)
def _edge_kernel(h_hbm, src_hbm, dst_hbm, zrows_hbm, out_hbm,
                 src_v, dst_v, rows_v, acc_sh, gsem):
    cid = lax.axis_index("c")
    sid = lax.axis_index("s")
    wid = sid * NC + cid
    rbase = sid * RPT
    pltpu.sync_copy(src_hbm.at[wid], src_v)
    pltpu.sync_copy(dst_hbm.at[wid], dst_v)
    pltpu.sync_copy(zrows_hbm, rows_v.at[pl.ds(0, ZC)])
    for t in range(RPT // ZC):
        pltpu.sync_copy(rows_v.at[pl.ds(0, ZC)],
                        acc_sh.at[pl.ds(rbase + t * ZC, ZC)])
    plsc.subcore_barrier()

    def chunk(j, carry):
        pltpu.async_copy(h_hbm.at[src_v.at[j]], rows_v, gsem).wait()
        pltpu.sync_copy(rows_v, acc_sh.at[dst_v.at[j]], add=True)
        return carry

    lax.fori_loop(0, EN, chunk, 0)
    plsc.subcore_barrier()
    for t in range(RPT // ZC):
        pltpu.sync_copy(acc_sh.at[pl.ds(rbase + t * ZC, ZC)],
                        rows_v.at[pl.ds(0, ZC)])
        pltpu.sync_copy(rows_v.at[pl.ds(0, ZC)],
                        out_hbm.at[cid, pl.ds(rbase + t * ZC, ZC)])


def _norm_body(p_ref, out_ref):
    deg = p_ref[0] + p_ref[1] + 1.0
    out_ref[...] = lax.rsqrt(deg)


def _mm_scale_body(x_ref, w_ref, s_ref, o_ref):
    h = jnp.dot(x_ref[...], w_ref[...], preferred_element_type=jnp.float32)
    o_ref[...] = h * s_ref[...]


def _combine_mm_body(p_ref, hp_ref, ni_ref, b_ref, w_ref, no_ref, o_ref):
    agg = p_ref[0] + p_ref[1] + hp_ref[...]
    h = jnp.maximum(agg * ni_ref[...] + b_ref[...], 0.0)
    o_ref[...] = jnp.dot(h, w_ref[...], preferred_element_type=jnp.float32) * no_ref[...]


def _combine_final_body(p_ref, hp_ref, ni_ref, b_ref, o_ref):
    agg = p_ref[0] + p_ref[1] + hp_ref[...]
    o_ref[...] = agg * ni_ref[...] + b_ref[...]


def _norms(deg_p):
    return pl.pallas_call(
        _norm_body,
        out_shape=jax.ShapeDtypeStruct((2, N_PAD), jnp.float32),
    )(deg_p)


def _mm_scale(xv, W, s_col):
    return pl.pallas_call(
        _mm_scale_body,
        grid=(N // BM,),
        in_specs=[
            pl.BlockSpec((BM, D), lambda i: (i, 0)),
            pl.BlockSpec((D, D), lambda i: (0, 0)),
            pl.BlockSpec((BM, 1), lambda i: (i, 0)),
        ],
        out_specs=pl.BlockSpec((BM, D), lambda i: (i, 0)),
        out_shape=jax.ShapeDtypeStruct((N, D), jnp.float32),
    )(xv, W, s_col)


def _combine_mm(part, hp, ni_col, b_row, W, no_col):
    return pl.pallas_call(
        _combine_mm_body,
        grid=(N // BM,),
        in_specs=[
            pl.BlockSpec((NC, BM, D), lambda i: (0, i, 0)),
            pl.BlockSpec((BM, D), lambda i: (i, 0)),
            pl.BlockSpec((BM, 1), lambda i: (i, 0)),
            pl.BlockSpec((1, D), lambda i: (0, 0)),
            pl.BlockSpec((D, D), lambda i: (0, 0)),
            pl.BlockSpec((BM, 1), lambda i: (i, 0)),
        ],
        out_specs=pl.BlockSpec((BM, D), lambda i: (i, 0)),
        out_shape=jax.ShapeDtypeStruct((N, D), jnp.float32),
    )(part, hp, ni_col, b_row, W, no_col)


def _combine_final(part, hp, ni_col, b_row):
    return pl.pallas_call(
        _combine_final_body,
        grid=(N // BM,),
        in_specs=[
            pl.BlockSpec((NC, BM, D), lambda i: (0, i, 0)),
            pl.BlockSpec((BM, D), lambda i: (i, 0)),
            pl.BlockSpec((BM, 1), lambda i: (i, 0)),
            pl.BlockSpec((1, D), lambda i: (0, 0)),
        ],
        out_specs=pl.BlockSpec((BM, D), lambda i: (i, 0)),
        out_shape=jax.ShapeDtypeStruct((N, D), jnp.float32),
    )(part, hp, ni_col, b_row)


def kernel(x, edge_index, W0, b0, W1, b1):
    src = edge_index[0]
    dst = edge_index[1]
    src_d = src.reshape(NW, DN, DC)
    dst_d = dst.reshape(NW, DN, DC)
    ppt = (E_PAD - E) // NW
    src_pad = jnp.zeros((NW, ppt), jnp.int32)
    dst_pad = jnp.broadcast_to(N + jnp.arange(ppt, dtype=jnp.int32), (NW, ppt))
    src_e = jnp.concatenate([src.reshape(NW, E // NW), src_pad], axis=1).reshape(NW, EN, EC)
    dst_e = jnp.concatenate([dst.reshape(NW, E // NW), dst_pad], axis=1).reshape(NW, EN, EC)
    ones_c = jnp.ones((DC,), jnp.float32)
    zeros_r = jnp.zeros((RPT,), jnp.float32)
    zrows = jnp.zeros((ZC, D), jnp.float32)

    deg_p = _deg_kernel(src_d, dst_d, ones_c, zeros_r)
    norms = _norms(deg_p)
    no_col = norms[0, :N].reshape(N, 1)
    ni_col = norms[1, :N].reshape(N, 1)

    h0p = _mm_scale(x, W0, no_col)
    part0 = _edge_kernel(h0p, src_e, dst_e, zrows)
    h1p = _combine_mm(part0, h0p, ni_col, b0.reshape(1, D), W1, no_col)
    part1 = _edge_kernel(h1p, src_e, dst_e, zrows)
    return _combine_final(part1, h1p, ni_col, b1.reshape(1, D))


# speedup vs baseline: 3.2209x; 1.5272x over previous
import functools

import jax
import jax.numpy as jnp
from jax import lax
from jax.experimental import pallas as pl
from jax.experimental.pallas import tpu as pltpu
from jax.experimental.pallas import tpu_sc as plsc

N = 10000
E = 320000
D = 128
N_PAD = 10240
NC = 2
NS = 16
NW = NC * NS
RPT = N_PAD // NS
BM = 1000

EC = 80
EN = 125
E_PAD = NW * EN * EC
ZC = 80

DC = 80
DN = E // NW // DC
NBUF = 2
DG = DN // NBUF

_mesh = plsc.VectorSubcoreMesh(core_axis_name="c", subcore_axis_name="s")


@functools.partial(
    pl.kernel,
    mesh=_mesh,
    out_type=jax.ShapeDtypeStruct((NC, 2, N_PAD), jnp.float32),
    scratch_types=[
        pltpu.VMEM((DN, DC), jnp.int32),
        pltpu.VMEM((DN, DC), jnp.int32),
        pltpu.VMEM((DC,), jnp.float32),
        pltpu.VMEM_SHARED((N_PAD,), jnp.float32),
        pltpu.VMEM_SHARED((N_PAD,), jnp.float32),
        pltpu.SemaphoreType.DMA((NBUF,)),
        pltpu.SemaphoreType.DMA((NBUF,)),
    ],
)
def _deg_kernel(src_hbm, dst_hbm, ones_hbm, zeros_hbm, out_hbm,
                src_v, dst_v, ones_v, dego_sh, degi_sh, osem, isem):
    cid = lax.axis_index("c")
    sid = lax.axis_index("s")
    wid = sid * NC + cid
    pltpu.sync_copy(src_hbm.at[wid], src_v)
    pltpu.sync_copy(dst_hbm.at[wid], dst_v)
    pltpu.sync_copy(ones_hbm, ones_v)
    pltpu.sync_copy(zeros_hbm, dego_sh.at[pl.ds(sid * RPT, RPT)])
    pltpu.sync_copy(zeros_hbm, degi_sh.at[pl.ds(sid * RPT, RPT)])
    plsc.subcore_barrier()

    def start(j, b):
        pltpu.async_copy(ones_v, dego_sh.at[src_v.at[j]], osem.at[b], add=True)
        pltpu.async_copy(ones_v, degi_sh.at[dst_v.at[j]], isem.at[b], add=True)

    def drain(j, b):
        pltpu.make_async_copy(ones_v, dego_sh.at[src_v.at[j]], osem.at[b]).wait()
        pltpu.make_async_copy(ones_v, degi_sh.at[dst_v.at[j]], isem.at[b]).wait()

    for b in range(NBUF):
        start(b, b)

    def group(g, carry):
        for b in range(NBUF):
            j = g * NBUF + b
            drain(j, b)
            start(j + NBUF, b)
        return carry

    lax.fori_loop(0, DG - 1, group, 0)
    for b in range(NBUF):
        drain((DG - 1) * NBUF + b, b)
    start(DN - 1, 0)
    drain(DN - 1, 0)
    plsc.subcore_barrier()
    pltpu.sync_copy(dego_sh.at[pl.ds(sid * RPT, RPT)],
                    out_hbm.at[cid, 0, pl.ds(sid * RPT, RPT)])
    pltpu.sync_copy(degi_sh.at[pl.ds(sid * RPT, RPT)],
                    out_hbm.at[cid, 1, pl.ds(sid * RPT, RPT)])


@functools.partial(
    pl.kernel,
    mesh=_mesh,
    out_type=jax.ShapeDtypeStruct((NC, N_PAD, D), jnp.float32),
    scratch_types=[
        pltpu.VMEM((EN, EC), jnp.int32),
        pltpu.VMEM((2, EC), jnp.int32),
        pltpu.VMEM((2, EC), jnp.int32),
        pltpu.VMEM((2, EC, D), jnp.float32),
        pltpu.VMEM_SHARED((N_PAD, D), jnp.float32),
        pltpu.SemaphoreType.DMA((2,)),
        pltpu.SemaphoreType.DMA((2,)),
    ],
)
def _edge_kernel(h_hbm, pk_hbm, zrows_hbm, out_hbm,
                 pk_v, srcb_v, dstb_v, rows_v, acc_sh, gsem, ssem):
    cid = lax.axis_index("c")
    sid = lax.axis_index("s")
    wid = sid * NC + cid
    rbase = sid * RPT
    pltpu.sync_copy(pk_hbm.at[wid], pk_v)
    pltpu.sync_copy(zrows_hbm, rows_v.at[0])
    for t in range(RPT // ZC):
        pltpu.sync_copy(rows_v.at[0], acc_sh.at[pl.ds(rbase + t * ZC, ZC)])
    plsc.subcore_barrier()

    mask = jnp.int32(0xFFFF)

    def unpack(j, b):
        for k in range(EC // 16):
            w = pk_v[j, pl.ds(k * 16, 16)]
            srcb_v[b, pl.ds(k * 16, 16)] = w & mask
            dstb_v[b, pl.ds(k * 16, 16)] = lax.shift_right_logical(w, 16)

    def g_start(b):
        pltpu.async_copy(h_hbm.at[srcb_v.at[b]], rows_v.at[b], gsem.at[b])

    def g_wait(b):
        pltpu.make_async_copy(h_hbm.at[srcb_v.at[b]], rows_v.at[b],
                              gsem.at[b]).wait()

    def s_start(b):
        pltpu.async_copy(rows_v.at[b], acc_sh.at[dstb_v.at[b]],
                         ssem.at[b], add=True)

    def s_wait(b):
        pltpu.make_async_copy(rows_v.at[b], acc_sh.at[dstb_v.at[b]],
                              ssem.at[b]).wait()

    unpack(0, 0)
    g_start(0)
    unpack(1, 1)
    g_start(1)

    def slot(j, b, issue_next):
        g_wait(b)
        s_start(b)
        s_wait(b)
        if issue_next:
            unpack(j + 2, b)
            g_start(b)

    def group(g, carry):
        slot(2 * g, 0, True)
        slot(2 * g + 1, 1, True)
        return carry

    lax.fori_loop(0, (EN - 3) // 2, group, 0)
    slot(EN - 3, 0, True)
    slot(EN - 2, 1, False)
    slot(EN - 1, 0, False)
    plsc.subcore_barrier()
    for t in range(RPT // ZC):
        pltpu.sync_copy(acc_sh.at[pl.ds(rbase + t * ZC, ZC)], rows_v.at[0])
        pltpu.sync_copy(rows_v.at[0],
                        out_hbm.at[cid, pl.ds(rbase + t * ZC, ZC)])


def _norm_body(p_ref, out_ref):
    deg = p_ref[0] + p_ref[1] + 1.0
    out_ref[...] = lax.rsqrt(deg)


def _mm_scale_body(x_ref, w_ref, s_ref, o_ref):
    h = jnp.dot(x_ref[...], w_ref[...], preferred_element_type=jnp.float32)
    o_ref[...] = h * s_ref[...]


def _combine_mm_body(p_ref, hp_ref, ni_ref, b_ref, w_ref, no_ref, o_ref):
    agg = p_ref[0] + p_ref[1] + hp_ref[...]
    h = jnp.maximum(agg * ni_ref[...] + b_ref[...], 0.0)
    o_ref[...] = jnp.dot(h, w_ref[...], preferred_element_type=jnp.float32) * no_ref[...]


def _combine_final_body(p_ref, hp_ref, ni_ref, b_ref, o_ref):
    agg = p_ref[0] + p_ref[1] + hp_ref[...]
    o_ref[...] = agg * ni_ref[...] + b_ref[...]


def _norms(deg_p):
    return pl.pallas_call(
        _norm_body,
        out_shape=jax.ShapeDtypeStruct((2, N_PAD), jnp.float32),
    )(deg_p)


def _mm_scale(xv, W, s_col):
    return pl.pallas_call(
        _mm_scale_body,
        grid=(N // BM,),
        in_specs=[
            pl.BlockSpec((BM, D), lambda i: (i, 0)),
            pl.BlockSpec((D, D), lambda i: (0, 0)),
            pl.BlockSpec((BM, 1), lambda i: (i, 0)),
        ],
        out_specs=pl.BlockSpec((BM, D), lambda i: (i, 0)),
        out_shape=jax.ShapeDtypeStruct((N, D), jnp.float32),
    )(xv, W, s_col)


def _combine_mm(part, hp, ni_col, b_row, W, no_col):
    return pl.pallas_call(
        _combine_mm_body,
        grid=(N // BM,),
        in_specs=[
            pl.BlockSpec((NC, BM, D), lambda i: (0, i, 0)),
            pl.BlockSpec((BM, D), lambda i: (i, 0)),
            pl.BlockSpec((BM, 1), lambda i: (i, 0)),
            pl.BlockSpec((1, D), lambda i: (0, 0)),
            pl.BlockSpec((D, D), lambda i: (0, 0)),
            pl.BlockSpec((BM, 1), lambda i: (i, 0)),
        ],
        out_specs=pl.BlockSpec((BM, D), lambda i: (i, 0)),
        out_shape=jax.ShapeDtypeStruct((N, D), jnp.float32),
    )(part, hp, ni_col, b_row, W, no_col)


def _combine_final(part, hp, ni_col, b_row):
    return pl.pallas_call(
        _combine_final_body,
        grid=(N // BM,),
        in_specs=[
            pl.BlockSpec((NC, BM, D), lambda i: (0, i, 0)),
            pl.BlockSpec((BM, D), lambda i: (i, 0)),
            pl.BlockSpec((BM, 1), lambda i: (i, 0)),
            pl.BlockSpec((1, D), lambda i: (0, 0)),
        ],
        out_specs=pl.BlockSpec((BM, D), lambda i: (i, 0)),
        out_shape=jax.ShapeDtypeStruct((N, D), jnp.float32),
    )(part, hp, ni_col, b_row)


def kernel(x, edge_index, W0, b0, W1, b1):
    src = edge_index[0]
    dst = edge_index[1]
    src_d = src.reshape(NW, DN, DC)
    dst_d = dst.reshape(NW, DN, DC)
    pk_e = ((dst << 16) | src).reshape(NW, EN, EC)
    ones_c = jnp.ones((DC,), jnp.float32)
    zeros_r = jnp.zeros((RPT,), jnp.float32)
    zrows = jnp.zeros((ZC, D), jnp.float32)

    deg_p = _deg_kernel(src_d, dst_d, ones_c, zeros_r)
    norms = _norms(deg_p)
    no_col = norms[0, :N].reshape(N, 1)
    ni_col = norms[1, :N].reshape(N, 1)

    h0p = _mm_scale(x, W0, no_col)
    part0 = _edge_kernel(h0p, pk_e, zrows)
    h1p = _combine_mm(part0, h0p, ni_col, b0.reshape(1, D), W1, no_col)
    part1 = _edge_kernel(h1p, pk_e, zrows)
    return _combine_final(part1, h1p, ni_col, b1.reshape(1, D))


# speedup vs baseline: 3.5879x; 1.1140x over previous
import functools

import jax
import jax.numpy as jnp
from jax import lax
from jax.experimental import pallas as pl
from jax.experimental.pallas import tpu as pltpu
from jax.experimental.pallas import tpu_sc as plsc

N = 10000
E = 320000
D = 128
N_PAD = 10240
NC = 2
NS = 16
NW = NC * NS
RPT = N_PAD // NS
BM = 1000

EC = 80
EN = 125
E_PAD = NW * EN * EC
ZC = 80

DC = 80
DN = E // NW // DC
NBUF = 2
DG = DN // NBUF

_mesh = plsc.VectorSubcoreMesh(core_axis_name="c", subcore_axis_name="s")


@functools.partial(
    pl.kernel,
    mesh=_mesh,
    out_type=jax.ShapeDtypeStruct((NC, 2, N_PAD), jnp.float32),
    scratch_types=[
        pltpu.VMEM((DN, DC), jnp.int32),
        pltpu.VMEM((DN, DC), jnp.int32),
        pltpu.VMEM((DC,), jnp.float32),
        pltpu.VMEM_SHARED((N_PAD,), jnp.float32),
        pltpu.VMEM_SHARED((N_PAD,), jnp.float32),
        pltpu.SemaphoreType.DMA((NBUF,)),
        pltpu.SemaphoreType.DMA((NBUF,)),
    ],
)
def _deg_kernel(src_hbm, dst_hbm, ones_hbm, zeros_hbm, out_hbm,
                src_v, dst_v, ones_v, dego_sh, degi_sh, osem, isem):
    cid = lax.axis_index("c")
    sid = lax.axis_index("s")
    wid = sid * NC + cid
    pltpu.sync_copy(src_hbm.at[wid], src_v)
    pltpu.sync_copy(dst_hbm.at[wid], dst_v)
    pltpu.sync_copy(ones_hbm, ones_v)
    pltpu.sync_copy(zeros_hbm, dego_sh.at[pl.ds(sid * RPT, RPT)])
    pltpu.sync_copy(zeros_hbm, degi_sh.at[pl.ds(sid * RPT, RPT)])
    plsc.subcore_barrier()

    def start(j, b):
        pltpu.async_copy(ones_v, dego_sh.at[src_v.at[j]], osem.at[b], add=True)
        pltpu.async_copy(ones_v, degi_sh.at[dst_v.at[j]], isem.at[b], add=True)

    def drain(j, b):
        pltpu.make_async_copy(ones_v, dego_sh.at[src_v.at[j]], osem.at[b]).wait()
        pltpu.make_async_copy(ones_v, degi_sh.at[dst_v.at[j]], isem.at[b]).wait()

    for b in range(NBUF):
        start(b, b)

    def group(g, carry):
        for b in range(NBUF):
            j = g * NBUF + b
            drain(j, b)
            start(j + NBUF, b)
        return carry

    lax.fori_loop(0, DG - 1, group, 0)
    for b in range(NBUF):
        drain((DG - 1) * NBUF + b, b)
    start(DN - 1, 0)
    drain(DN - 1, 0)
    plsc.subcore_barrier()
    pltpu.sync_copy(dego_sh.at[pl.ds(sid * RPT, RPT)],
                    out_hbm.at[cid, 0, pl.ds(sid * RPT, RPT)])
    pltpu.sync_copy(degi_sh.at[pl.ds(sid * RPT, RPT)],
                    out_hbm.at[cid, 1, pl.ds(sid * RPT, RPT)])


@functools.partial(
    pl.kernel,
    mesh=_mesh,
    out_type=jax.ShapeDtypeStruct((NC, N_PAD, D), jnp.float32),
    scratch_types=[
        pltpu.VMEM((6, 1, EC), jnp.int32),
        pltpu.VMEM((4, EC), jnp.int32),
        pltpu.VMEM((4, EC), jnp.int32),
        pltpu.VMEM((3, EC, D), jnp.float32),
        pltpu.VMEM_SHARED((N_PAD, D), jnp.float32),
        pltpu.SemaphoreType.DMA((3,)),
        pltpu.SemaphoreType.DMA((3,)),
        pltpu.SemaphoreType.DMA((6,)),
    ],
)
def _edge_kernel(h_hbm, pk_hbm, zrows_hbm, out_hbm,
                 pk_v, srcb_v, dstb_v, rows_v, acc_sh, gsem, ssem, psem):
    cid = lax.axis_index("c")
    sid = lax.axis_index("s")
    wid = sid * NC + cid
    rbase = sid * RPT
    pltpu.sync_copy(zrows_hbm, rows_v.at[0])
    for t in range(RPT // ZC):
        pltpu.sync_copy(rows_v.at[0], acc_sh.at[pl.ds(rbase + t * ZC, ZC)])
    plsc.subcore_barrier()

    mask = jnp.int32(0xFFFF)

    def pk_load(j):
        pltpu.async_copy(pk_hbm.at[wid, j], pk_v.at[j % 6], psem.at[j % 6])

    def pk_wait(j):
        pltpu.make_async_copy(pk_hbm.at[wid, j], pk_v.at[j % 6],
                              psem.at[j % 6]).wait()

    def unpack(j, p6, b4):
        pk_wait(j)
        for k in range(EC // 16):
            w = pk_v[p6, 0, pl.ds(k * 16, 16)]
            srcb_v[b4, pl.ds(k * 16, 16)] = w & mask
            dstb_v[b4, pl.ds(k * 16, 16)] = lax.shift_right_logical(w, 16)

    def g_start(b3, b4):
        pltpu.async_copy(h_hbm.at[srcb_v.at[b4]], rows_v.at[b3], gsem.at[b3])

    def g_wait(b3, b4):
        pltpu.make_async_copy(h_hbm.at[srcb_v.at[b4]], rows_v.at[b3],
                              gsem.at[b3]).wait()

    def s_start(b3, b4):
        pltpu.async_copy(rows_v.at[b3], acc_sh.at[dstb_v.at[b4]],
                         ssem.at[b3], add=True)

    def s_wait(b3, b4):
        pltpu.make_async_copy(rows_v.at[b3], acc_sh.at[dstb_v.at[b4]],
                              ssem.at[b3]).wait()

    def slot(j, u, pk6=True, up4=True, g2=True, sw1=True):
        if pk6:
            pk_load(j + 6)
        g_wait(u % 3, u % 4)
        s_start(u % 3, u % 4)
        if sw1:
            s_wait((u - 1) % 3, (u - 1) % 4)
        if up4:
            unpack(j + 3, (u + 3) % 6, (u + 3) % 4)
        if g2:
            g_start((u + 2) % 3, (u + 2) % 4)

    for j in range(6):
        pk_load(j)
    for j in range(3):
        unpack(j, j, j)
    g_start(0, 0)
    g_start(1, 1)
    slot(0, 0, sw1=False)
    for j in range(1, 12):
        slot(j, j)

    def group(g, carry):
        base = 12 * g
        for u in range(12):
            slot(base + u, u)
        return carry

    lax.fori_loop(1, 9, group, 0)
    for j in range(108, EN):
        slot(j, j % 12, pk6=(j + 6 < EN), up4=(j + 3 < EN), g2=(j + 2 < EN),
             sw1=True)
    s_wait((EN - 1) % 3, (EN - 1) % 4)
    plsc.subcore_barrier()
    for t in range(RPT // ZC):
        pltpu.sync_copy(acc_sh.at[pl.ds(rbase + t * ZC, ZC)], rows_v.at[0])
        pltpu.sync_copy(rows_v.at[0],
                        out_hbm.at[cid, pl.ds(rbase + t * ZC, ZC)])


def _norm_body(p_ref, out_ref):
    deg = p_ref[0] + p_ref[1] + 1.0
    out_ref[...] = lax.rsqrt(deg)


def _mm_scale_body(x_ref, w_ref, s_ref, o_ref):
    h = jnp.dot(x_ref[...], w_ref[...], preferred_element_type=jnp.float32)
    o_ref[...] = h * s_ref[...]


def _combine_mm_body(p_ref, hp_ref, ni_ref, b_ref, w_ref, no_ref, o_ref):
    agg = p_ref[0] + p_ref[1] + hp_ref[...]
    h = jnp.maximum(agg * ni_ref[...] + b_ref[...], 0.0)
    o_ref[...] = jnp.dot(h, w_ref[...], preferred_element_type=jnp.float32) * no_ref[...]


def _combine_final_body(p_ref, hp_ref, ni_ref, b_ref, o_ref):
    agg = p_ref[0] + p_ref[1] + hp_ref[...]
    o_ref[...] = agg * ni_ref[...] + b_ref[...]


def _norms(deg_p):
    return pl.pallas_call(
        _norm_body,
        out_shape=jax.ShapeDtypeStruct((2, N_PAD), jnp.float32),
    )(deg_p)


def _mm_scale(xv, W, s_col):
    return pl.pallas_call(
        _mm_scale_body,
        grid=(N // BM,),
        in_specs=[
            pl.BlockSpec((BM, D), lambda i: (i, 0)),
            pl.BlockSpec((D, D), lambda i: (0, 0)),
            pl.BlockSpec((BM, 1), lambda i: (i, 0)),
        ],
        out_specs=pl.BlockSpec((BM, D), lambda i: (i, 0)),
        out_shape=jax.ShapeDtypeStruct((N, D), jnp.float32),
    )(xv, W, s_col)


def _combine_mm(part, hp, ni_col, b_row, W, no_col):
    return pl.pallas_call(
        _combine_mm_body,
        grid=(N // BM,),
        in_specs=[
            pl.BlockSpec((NC, BM, D), lambda i: (0, i, 0)),
            pl.BlockSpec((BM, D), lambda i: (i, 0)),
            pl.BlockSpec((BM, 1), lambda i: (i, 0)),
            pl.BlockSpec((1, D), lambda i: (0, 0)),
            pl.BlockSpec((D, D), lambda i: (0, 0)),
            pl.BlockSpec((BM, 1), lambda i: (i, 0)),
        ],
        out_specs=pl.BlockSpec((BM, D), lambda i: (i, 0)),
        out_shape=jax.ShapeDtypeStruct((N, D), jnp.float32),
    )(part, hp, ni_col, b_row, W, no_col)


def _combine_final(part, hp, ni_col, b_row):
    return pl.pallas_call(
        _combine_final_body,
        grid=(N // BM,),
        in_specs=[
            pl.BlockSpec((NC, BM, D), lambda i: (0, i, 0)),
            pl.BlockSpec((BM, D), lambda i: (i, 0)),
            pl.BlockSpec((BM, 1), lambda i: (i, 0)),
            pl.BlockSpec((1, D), lambda i: (0, 0)),
        ],
        out_specs=pl.BlockSpec((BM, D), lambda i: (i, 0)),
        out_shape=jax.ShapeDtypeStruct((N, D), jnp.float32),
    )(part, hp, ni_col, b_row)


def kernel(x, edge_index, W0, b0, W1, b1):
    src = edge_index[0]
    dst = edge_index[1]
    src_d = src.reshape(NW, DN, DC)
    dst_d = dst.reshape(NW, DN, DC)
    pk_e = ((dst << 16) | src).reshape(NW, EN, 1, EC)
    ones_c = jnp.ones((DC,), jnp.float32)
    zeros_r = jnp.zeros((RPT,), jnp.float32)
    zrows = jnp.zeros((ZC, D), jnp.float32)

    deg_p = _deg_kernel(src_d, dst_d, ones_c, zeros_r)
    norms = _norms(deg_p)
    no_col = norms[0, :N].reshape(N, 1)
    ni_col = norms[1, :N].reshape(N, 1)

    h0p = _mm_scale(x, W0, no_col)
    part0 = _edge_kernel(h0p, pk_e, zrows)
    h1p = _combine_mm(part0, h0p, ni_col, b0.reshape(1, D), W1, no_col)
    part1 = _edge_kernel(h1p, pk_e, zrows)
    return _combine_final(part1, h1p, ni_col, b1.reshape(1, D))
